# baseline (speedup 1.0000x reference)
import functools

import jax
import jax.numpy as jnp
from jax import lax
from jax.experimental import pallas as pl
from jax.experimental.pallas import tpu as pltpu
from jax.experimental.pallas import tpu_sc as plsc

VOCAB = 100000
D = 300
BATCH = 4096
SEQ = 50
B_TOT = BATCH * SEQ
NC = 2
NS = 16
NW = NC * NS
B_PER_W = B_TOT // NW
CHUNK = 128
N_CHUNK = B_PER_W // CHUNK
T2_OFF = 256
T2_W = D - T2_OFF

_mesh = plsc.VectorSubcoreMesh(core_axis_name="c", subcore_axis_name="s")


@functools.partial(
    pl.kernel,
    mesh=_mesh,
    compiler_params=pltpu.CompilerParams(use_tc_tiling_on_sc=True),
    out_type=jax.ShapeDtypeStruct((B_TOT, D), jnp.float32),
    scratch_types=[
        pltpu.VMEM((N_CHUNK, CHUNK), jnp.int32),
        pltpu.VMEM((CHUNK, D), jnp.float32),
        pltpu.VMEM((CHUNK, D), jnp.float32),
        pltpu.SemaphoreType.DMA,
        pltpu.SemaphoreType.DMA,
    ],
)
def _gather_rows(table_hbm, tail_hbm, idx_hbm, out_hbm,
                 idx_v, asm_a, asm_b, sem_a, sem_b):
    wid = lax.axis_index("s") * NC + lax.axis_index("c")
    base = wid * B_PER_W
    pltpu.sync_copy(idx_hbm.at[wid], idx_v)

    def start(j, buf, sem):
        ids = idx_v.at[j]
        pltpu.async_copy(table_hbm.at[ids, pl.ds(0, 128)],
                         buf.at[:, pl.ds(0, 128)], sem)
        pltpu.async_copy(table_hbm.at[ids, pl.ds(128, 128)],
                         buf.at[:, pl.ds(128, 128)], sem)

    def finish(j, buf, sem):
        rows = pl.ds(base + j * CHUNK, CHUNK)
        ids = idx_v.at[j]
        pltpu.make_async_copy(table_hbm.at[ids, pl.ds(0, 128)],
                              buf.at[:, pl.ds(0, 128)], sem).wait()
        pltpu.make_async_copy(table_hbm.at[ids, pl.ds(128, 128)],
                              buf.at[:, pl.ds(128, 128)], sem).wait()
        pltpu.sync_copy(tail_hbm.at[rows], buf.at[:, pl.ds(T2_OFF, T2_W)])
        pltpu.sync_copy(buf, out_hbm.at[rows])

    start(0, asm_a, sem_a)

    def body(p, carry):
        j0 = 2 * p
        start(j0 + 1, asm_b, sem_b)
        finish(j0, asm_a, sem_a)

        @pl.when(p < N_CHUNK // 2 - 1)
        def _():
            start(j0 + 2, asm_a, sem_a)

        finish(j0 + 1, asm_b, sem_b)
        return carry

    lax.fori_loop(0, N_CHUNK // 2, body, 0)


def kernel(word_sequences, table):
    idx_flat = word_sequences.astype(jnp.int32).reshape(B_TOT)
    idx = idx_flat.reshape(NW, N_CHUNK, CHUNK)
    t2p = jnp.pad(lax.slice(table, (0, T2_OFF), (VOCAB, D)),
                  ((0, 0), (0, 128 - T2_W)))
    tail = lax.slice(jnp.take(t2p, word_sequences, axis=0)
                     .reshape(B_TOT, 128), (0, 0), (B_TOT, T2_W))
    out = _gather_rows(table, tail, idx)
    return out.reshape(BATCH, SEQ, D)


# speedup vs baseline: 1.5925x; 1.5925x over previous
import functools

import jax
import jax.numpy as jnp
from jax import lax
from jax.experimental import pallas as pl
from jax.experimental.pallas import tpu as pltpu
from jax.experimental.pallas import tpu_sc as plsc

VOCAB = 100000
D = 300
BATCH = 4096
SEQ = 50
B_TOT = BATCH * SEQ
NC = 2
NS = 16
NW = NC * NS
B_PER_W = B_TOT // NW
CHUNK = 128
N_CHUNK = B_PER_W // CHUNK
T2_OFF = 256
T2_W = D - T2_OFF

_mesh = plsc.VectorSubcoreMesh(core_axis_name="c", subcore_axis_name="s")


@functools.partial(
    pl.kernel,
    mesh=_mesh,
    compiler_params=pltpu.CompilerParams(use_tc_tiling_on_sc=True),
    out_type=jax.ShapeDtypeStruct((B_TOT, D), jnp.float32),
    scratch_types=[
        pltpu.VMEM((N_CHUNK, CHUNK), jnp.int32),
        pltpu.VMEM((CHUNK, D), jnp.float32),
        pltpu.VMEM((CHUNK, D), jnp.float32),
        pltpu.SemaphoreType.DMA,
        pltpu.SemaphoreType.DMA,
    ],
)
def _gather_rows(table_hbm, tail_hbm, idx_hbm, out_hbm,
                 idx_v, asm_a, asm_b, sem_a, sem_b):
    wid = lax.axis_index("s") * NC + lax.axis_index("c")
    base = wid * B_PER_W
    pltpu.sync_copy(idx_hbm.at[wid], idx_v)

    def start(j, buf, sem):
        ids = idx_v.at[j]
        pltpu.async_copy(table_hbm.at[ids, pl.ds(0, 128)],
                         buf.at[:, pl.ds(0, 128)], sem)
        pltpu.async_copy(table_hbm.at[ids, pl.ds(128, 128)],
                         buf.at[:, pl.ds(128, 128)], sem)

    def finish(j, buf, sem):
        rows = pl.ds(base + j * CHUNK, CHUNK)
        ids = idx_v.at[j]
        pltpu.make_async_copy(table_hbm.at[ids, pl.ds(0, 128)],
                              buf.at[:, pl.ds(0, 128)], sem).wait()
        pltpu.make_async_copy(table_hbm.at[ids, pl.ds(128, 128)],
                              buf.at[:, pl.ds(128, 128)], sem).wait()
        pltpu.sync_copy(tail_hbm.at[rows], buf.at[:, pl.ds(T2_OFF, T2_W)])
        pltpu.sync_copy(buf, out_hbm.at[rows])

    start(0, asm_a, sem_a)

    def body(p, carry):
        j0 = 2 * p
        start(j0 + 1, asm_b, sem_b)
        finish(j0, asm_a, sem_a)

        @pl.when(p < N_CHUNK // 2 - 1)
        def _():
            start(j0 + 2, asm_a, sem_a)

        finish(j0 + 1, asm_b, sem_b)
        return carry

    lax.fori_loop(0, N_CHUNK // 2, body, 0)


def kernel(word_sequences, table):
    idx_flat = word_sequences.astype(jnp.int32).reshape(B_TOT)
    idx = idx_flat.reshape(NW, N_CHUNK, CHUNK)
    t2p = jnp.pad(lax.slice(table, (0, T2_OFF), (VOCAB, D)),
                  ((0, 0), (0, 128 - T2_W)))
    tail = lax.slice(t2p.at[idx_flat].get(mode="promise_in_bounds"),
                     (0, 0), (B_TOT, T2_W))
    out = _gather_rows(table, tail, idx)
    return out.reshape(BATCH, SEQ, D)
